# baseline (speedup 1.0000x reference)
import functools

import jax
import jax.numpy as jnp
from jax.experimental import pallas as pl
from jax.experimental.pallas import tpu as pltpu


def _moe_kernel(x_ref, gw_ref, wg_ref, wu_ref, wd_ref, out_ref, comb_ref,
                *, n_experts):
    e = pl.program_id(0)
    i = pl.program_id(1)

    @pl.when((e == 0) & (i == 0))
    def _router():
        x = x_ref[...]
        logits = jnp.dot(x, gw_ref[...].T, preferred_element_type=jnp.float32)
        tb = logits.shape[0]
        idx = jax.lax.broadcasted_iota(jnp.int32, (tb, n_experts), 1)
        l1 = jnp.max(logits, axis=-1, keepdims=True)
        i1 = jnp.min(jnp.where(logits == l1, idx, n_experts), axis=-1,
                     keepdims=True)
        masked = jnp.where(idx == i1, -jnp.inf, logits)
        l2 = jnp.max(masked, axis=-1, keepdims=True)
        i2 = jnp.min(jnp.where(masked == l2, idx, n_experts), axis=-1,
                     keepdims=True)
        w1 = jax.nn.sigmoid(l1 - l2)
        w2 = 1.0 - w1
        comb_ref[...] = jnp.where(idx == i1, w1, 0.0) + jnp.where(idx == i2, w2, 0.0)

    x = x_ref[...]
    onehot = (jax.lax.broadcasted_iota(jnp.int32, (1, n_experts), 1) == e)
    coef = jnp.sum(jnp.where(onehot, comb_ref[...], 0.0), axis=-1,
                   keepdims=True)

    g = jnp.dot(x, wg_ref[0], preferred_element_type=jnp.float32)
    u = jnp.dot(x, wu_ref[0], preferred_element_type=jnp.float32)
    inter = g * jax.nn.sigmoid(g) * u * coef
    contrib = jnp.dot(inter, wd_ref[0], preferred_element_type=jnp.float32)

    @pl.when((e == 0) & (i == 0))
    def _store():
        out_ref[...] = contrib

    @pl.when((e > 0) | (i > 0))
    def _accum():
        out_ref[...] += contrib


def kernel(hidden_states, gate_proj_w, gate_weights, up_weights, down_weights):
    T, H = hidden_states.shape
    E, _, I = gate_weights.shape
    n_i = 2
    Ib = I // n_i
    grid = (E, n_i)

    return pl.pallas_call(
        functools.partial(_moe_kernel, n_experts=E),
        grid=grid,
        in_specs=[
            pl.BlockSpec((T, H), lambda e, i: (0, 0)),
            pl.BlockSpec((E, H), lambda e, i: (0, 0)),
            pl.BlockSpec((1, H, Ib), lambda e, i: (e, 0, i)),
            pl.BlockSpec((1, H, Ib), lambda e, i: (e, 0, i)),
            pl.BlockSpec((1, Ib, H), lambda e, i: (e, i, 0)),
        ],
        out_specs=pl.BlockSpec((T, H), lambda e, i: (0, 0)),
        out_shape=jax.ShapeDtypeStruct((T, H), hidden_states.dtype),
        scratch_shapes=[pltpu.VMEM((T, E), jnp.float32)],
        compiler_params=pltpu.CompilerParams(
            dimension_semantics=("arbitrary", "arbitrary"),
        ),
    )(hidden_states, gate_proj_w, gate_weights, up_weights, down_weights)


# speedup vs baseline: 1.0862x; 1.0862x over previous
import functools

import jax
import jax.numpy as jnp
from jax.experimental import pallas as pl
from jax.experimental.pallas import tpu as pltpu


def _moe_kernel(x_ref, gw_ref, wg_ref, wu_ref, wd_ref, out_ref, comb_ref,
                *, n_experts):
    e = pl.program_id(0)
    i = pl.program_id(1)

    @pl.when((e == 0) & (i == 0))
    def _router():
        x = x_ref[...]
        logits = jnp.dot(x, gw_ref[...].T, preferred_element_type=jnp.float32)
        tb = logits.shape[0]
        idx = jax.lax.broadcasted_iota(jnp.int32, (tb, n_experts), 1)
        l1 = jnp.max(logits, axis=-1, keepdims=True)
        i1 = jnp.min(jnp.where(logits == l1, idx, n_experts), axis=-1,
                     keepdims=True)
        masked = jnp.where(idx == i1, -jnp.inf, logits)
        l2 = jnp.max(masked, axis=-1, keepdims=True)
        i2 = jnp.min(jnp.where(masked == l2, idx, n_experts), axis=-1,
                     keepdims=True)
        w1 = jax.nn.sigmoid(l1 - l2)
        w2 = 1.0 - w1
        comb_ref[...] = jnp.where(idx == i1, w1, 0.0) + jnp.where(idx == i2, w2, 0.0)
        out_ref[...] = jnp.zeros_like(out_ref)

    x = x_ref[...]
    onehot = (jax.lax.broadcasted_iota(jnp.int32, (1, n_experts), 1) == e)
    coef = jnp.sum(jnp.where(onehot, comb_ref[...], 0.0), axis=-1,
                   keepdims=True)

    g = jnp.dot(x, wg_ref[0], preferred_element_type=jnp.float32)
    u = jnp.dot(x, wu_ref[0], preferred_element_type=jnp.float32)
    inter = g * jax.nn.sigmoid(g) * u * coef
    out_ref[...] += jnp.dot(inter, wd_ref[0], preferred_element_type=jnp.float32)


def kernel(hidden_states, gate_proj_w, gate_weights, up_weights, down_weights):
    T, H = hidden_states.shape
    E, _, I = gate_weights.shape
    n_i = 2
    Ib = I // n_i
    grid = (E, n_i)

    return pl.pallas_call(
        functools.partial(_moe_kernel, n_experts=E),
        grid=grid,
        in_specs=[
            pl.BlockSpec((T, H), lambda e, i: (0, 0)),
            pl.BlockSpec((E, H), lambda e, i: (0, 0)),
            pl.BlockSpec((1, H, Ib), lambda e, i: (e, 0, i)),
            pl.BlockSpec((1, H, Ib), lambda e, i: (e, 0, i)),
            pl.BlockSpec((1, Ib, H), lambda e, i: (e, i, 0)),
        ],
        out_specs=pl.BlockSpec((T, H), lambda e, i: (0, 0)),
        out_shape=jax.ShapeDtypeStruct((T, H), hidden_states.dtype),
        scratch_shapes=[pltpu.VMEM((T, E), jnp.float32)],
        compiler_params=pltpu.CompilerParams(
            dimension_semantics=("arbitrary", "arbitrary"),
        ),
    )(hidden_states, gate_proj_w, gate_weights, up_weights, down_weights)
